# baseline (speedup 1.0000x reference)
import jax
import jax.numpy as jnp
import numpy as np
from jax.experimental import pallas as pl
from jax.experimental.pallas import tpu as pltpu

F32 = jnp.float32
L = 2048
D = 64
H = 16
U = 40
NEG = -3e38
KC = 256
NKC = L // KC
QT = 128
NQT = L // QT

_IDX = np.asarray(jax.random.randint(jax.random.key(42), (L, U), 0, L))
_WCNT = np.zeros((L, L), np.float32)
np.add.at(_WCNT, (_IDX.ravel(), np.repeat(np.arange(L), U)), 1.0)
_WMASK = np.where(_WCNT > 0, np.float32(0), np.float32(NEG))


def _dot_tt(a, b):
    return jax.lax.dot_general(a, b, (((1,), (1,)), ((), ())),
                               preferred_element_type=F32)


def _m_kernel(q_ref, k_ref, wmask_ref, wcnt_ref, sel_ref, msc_ref, kscr_ref):
    p = pl.program_id(0)
    for hh in range(2):
        for c in range(NKC):
            kscr_ref[hh, c * KC:(c + 1) * KC, :] = (
                k_ref[c * KC:(c + 1) * KC, hh * D:(hh + 1) * D])
    for hh in range(2):
        mtiles = []
        for t in range(NQT):
            qt = q_ref[t * QT:(t + 1) * QT, hh * D:(hh + 1) * D]
            mmax = jnp.full((1, QT), NEG, F32)
            msum = jnp.zeros((1, QT), F32)
            for c in range(NKC):
                ks = kscr_ref[hh, c * KC:(c + 1) * KC, :]
                st = _dot_tt(ks, qt)
                wm = wmask_ref[c * KC:(c + 1) * KC, t * QT:(t + 1) * QT]
                wc = wcnt_ref[c * KC:(c + 1) * KC, t * QT:(t + 1) * QT]
                mmax = jnp.maximum(mmax,
                                   jnp.max(st + wm, axis=0, keepdims=True))
                msum = msum + jnp.sum(st * wc, axis=0, keepdims=True)
            mtiles.append(mmax - msum * (1.0 / L))
        mrow = jnp.concatenate(mtiles, axis=1)
        for pp in range(H // 2):
            @pl.when(p == pp)
            def _(pp=pp, hh=hh, mrow=mrow):
                msc_ref[2 * pp + hh:2 * pp + hh + 1, :] = mrow

    @pl.when(p == H // 2 - 1)
    def _():
        m = msc_ref[:, :]
        col = jax.lax.broadcasted_iota(jnp.int32, (H, L), 1)
        sel = jnp.zeros((H, 128), jnp.int32)
        lane = jax.lax.broadcasted_iota(jnp.int32, (H, 128), 1)
        for i in range(U):
            rowmax = jnp.max(m, axis=1, keepdims=True)
            qidx = jnp.min(jnp.where(m == rowmax, col, jnp.int32(1 << 30)),
                           axis=1, keepdims=True)
            sel = jnp.where(lane == i, jnp.broadcast_to(qidx, (H, 128)), sel)
            m = jnp.where(col == qidx, NEG, m)
        sel_ref[:, :] = sel


def _attn_kernel(sel_ref, q_ref, k_ref, v_ref, out_ref,
                 qsel_ref, s_ref, kscr_ref, vscr_ref):
    p = pl.program_id(0)
    scale = F32(1.0 / np.sqrt(D))

    for hh in range(2):
        for c in range(NKC):
            kscr_ref[hh, c * KC:(c + 1) * KC, :] = (
                k_ref[c * KC:(c + 1) * KC, hh * D:(hh + 1) * D])
            vscr_ref[hh, c * KC:(c + 1) * KC, :] = (
                v_ref[c * KC:(c + 1) * KC, hh * D:(hh + 1) * D])

    upds = []
    idxs_pair = []
    for hh in range(2):
        idxs = [sel_ref[2 * p + hh, i] for i in range(U)]
        idxs_pair.append(idxs)
        for i in range(U):
            qsel_ref[i:i + 1, :] = q_ref[pl.ds(idxs[i], 1),
                                         hh * D:(hh + 1) * D]
        qsel = qsel_ref[:, :]

        rowmax = jnp.full((U, 1), NEG, F32)
        for c in range(NKC):
            ks = kscr_ref[hh, c * KC:(c + 1) * KC, :]
            sc = _dot_tt(qsel, ks) * scale
            s_ref[:, c * KC:(c + 1) * KC] = sc
            rowmax = jnp.maximum(rowmax, jnp.max(sc, axis=1, keepdims=True))
        rowsum = jnp.zeros((U, 1), F32)
        upd = jnp.zeros((U, D), F32)
        for c in range(NKC):
            pr = jnp.exp(s_ref[:, c * KC:(c + 1) * KC] - rowmax)
            rowsum = rowsum + jnp.sum(pr, axis=1, keepdims=True)
            upd = upd + jax.lax.dot_general(
                pr, vscr_ref[hh, c * KC:(c + 1) * KC, :],
                (((1,), (0,)), ((), ())), preferred_element_type=F32)
        upds.append(upd / rowsum)

    acc = jnp.zeros((1, 2 * D), F32)
    for c in range(NKC):
        acc = acc + jnp.sum(v_ref[c * KC:(c + 1) * KC, :], axis=0,
                            keepdims=True)
    meanv = acc * (1.0 / L)
    for c in range(NKC):
        out_ref[c * KC:(c + 1) * KC, :] = jnp.broadcast_to(meanv, (KC, 2 * D))
    for hh in range(2):
        for i in range(U):
            row = out_ref[pl.ds(idxs_pair[hh][i], 1), :]
            ui = upds[hh][i:i + 1, :]
            if hh == 0:
                new = jnp.concatenate([ui, row[:, D:]], axis=1)
            else:
                new = jnp.concatenate([row[:, :D], ui], axis=1)
            out_ref[pl.ds(idxs_pair[hh][i], 1), :] = new


def kernel(queries, keys, values, attn_mask):
    B, Lq, Hh, Dd = queries.shape
    qf = queries.reshape(Lq, Hh * Dd)
    kf = keys.reshape(Lq, Hh * Dd)
    vf = values.reshape(Lq, Hh * Dd)

    wmask = jnp.asarray(_WMASK)
    wcnt = jnp.asarray(_WCNT)

    sel = pl.pallas_call(
        _m_kernel,
        grid=(Hh // 2,),
        in_specs=[
            pl.BlockSpec((L, 2 * D), lambda p: (0, p)),
            pl.BlockSpec((L, 2 * D), lambda p: (0, p)),
            pl.BlockSpec((L, L), lambda p: (0, 0)),
            pl.BlockSpec((L, L), lambda p: (0, 0)),
        ],
        out_specs=pl.BlockSpec((Hh, 128), lambda p: (0, 0)),
        out_shape=jax.ShapeDtypeStruct((Hh, 128), jnp.int32),
        scratch_shapes=[pltpu.VMEM((Hh, L), F32),
                        pltpu.VMEM((2, L, D), F32)],
    )(qf, kf, wmask, wcnt)

    ctx = pl.pallas_call(
        _attn_kernel,
        grid=(Hh // 2,),
        in_specs=[
            pl.BlockSpec(memory_space=pltpu.SMEM),
            pl.BlockSpec((L, 2 * D), lambda p: (0, p)),
            pl.BlockSpec((L, 2 * D), lambda p: (0, p)),
            pl.BlockSpec((L, 2 * D), lambda p: (0, p)),
        ],
        out_specs=pl.BlockSpec((L, 2 * D), lambda p: (0, p)),
        out_shape=jax.ShapeDtypeStruct((L, Hh * Dd), F32),
        scratch_shapes=[
            pltpu.VMEM((U, D), F32),
            pltpu.VMEM((U, L), F32),
            pltpu.VMEM((2, L, D), F32),
            pltpu.VMEM((2, L, D), F32),
        ],
    )(sel, qf, kf, vf)

    return ctx.reshape(B, Lq, Hh, Dd)


# speedup vs baseline: 1.0229x; 1.0229x over previous
import jax
import jax.numpy as jnp
import numpy as np
from jax.experimental import pallas as pl
from jax.experimental.pallas import tpu as pltpu

F32 = jnp.float32
L = 2048
D = 64
H = 16
U = 40
NEG = -3e38
KC = 256
NKC = L // KC
QT = 128
NQT = L // QT

_IDX = np.asarray(jax.random.randint(jax.random.key(42), (L, U), 0, L))
_WCNT = np.zeros((L, L), np.float32)
np.add.at(_WCNT, (_IDX.ravel(), np.repeat(np.arange(L), U)), 1.0)
_WMASK = np.where(_WCNT > 0, np.float32(0), np.float32(NEG))


def _dot_tt(a, b):
    return jax.lax.dot_general(a, b, (((1,), (1,)), ((), ())),
                               preferred_element_type=F32)


def _m_kernel(q_ref, k_ref, wmask_ref, wcnt_ref, sel_ref, msc_ref, kscr_ref):
    p = pl.program_id(0)
    for hh in range(2):
        for c in range(NKC):
            kscr_ref[hh, c * KC:(c + 1) * KC, :] = (
                k_ref[c * KC:(c + 1) * KC, hh * D:(hh + 1) * D])
    for hh in range(2):
        mtiles = []
        for t in range(NQT):
            qt = q_ref[t * QT:(t + 1) * QT, hh * D:(hh + 1) * D]
            mmax = jnp.full((1, QT), NEG, F32)
            msum = jnp.zeros((1, QT), F32)
            for c in range(NKC):
                ks = kscr_ref[hh, c * KC:(c + 1) * KC, :]
                st = _dot_tt(ks, qt)
                wm = wmask_ref[c * KC:(c + 1) * KC, t * QT:(t + 1) * QT]
                wc = wcnt_ref[c * KC:(c + 1) * KC, t * QT:(t + 1) * QT]
                mmax = jnp.maximum(mmax,
                                   jnp.max(st + wm, axis=0, keepdims=True))
                msum = msum + jnp.sum(st * wc, axis=0, keepdims=True)
            mtiles.append(mmax - msum * (1.0 / L))
        mrow = jnp.concatenate(mtiles, axis=1)
        for pp in range(H // 2):
            @pl.when(p == pp)
            def _(pp=pp, hh=hh, mrow=mrow):
                msc_ref[2 * pp + hh:2 * pp + hh + 1, :] = mrow

    @pl.when(p == H // 2 - 1)
    def _():
        m = msc_ref[:, :]
        col = jax.lax.broadcasted_iota(jnp.int32, (H, L), 1)
        sel = jnp.zeros((H, 128), jnp.int32)
        lane = jax.lax.broadcasted_iota(jnp.int32, (H, 128), 1)
        for i in range(U):
            rowmax = jnp.max(m, axis=1, keepdims=True)
            qidx = jnp.min(jnp.where(m == rowmax, col, jnp.int32(1 << 30)),
                           axis=1, keepdims=True)
            sel = jnp.where(lane == i, jnp.broadcast_to(qidx, (H, 128)), sel)
            m = jnp.where(col == qidx, NEG, m)
        sel_ref[:, :] = sel


def _attn_kernel(sel_ref, q_ref, k_ref, v_ref, out_ref, qsel_ref, s_ref):
    p = pl.program_id(0)
    scale = F32(1.0 / np.sqrt(D))

    upds = []
    idxs_pair = []
    for hh in range(2):
        idxs = [sel_ref[2 * p + hh, i] for i in range(U)]
        idxs_pair.append(idxs)
        for i in range(U):
            qsel_ref[i:i + 1, :] = q_ref[pl.ds(idxs[i], 1),
                                         hh * D:(hh + 1) * D]
        qsel = qsel_ref[:, :]

        rowmax = jnp.full((U, 1), NEG, F32)
        for c in range(NKC):
            ks = k_ref[c * KC:(c + 1) * KC, hh * D:(hh + 1) * D]
            sc = _dot_tt(qsel, ks) * scale
            s_ref[:, c * KC:(c + 1) * KC] = sc
            rowmax = jnp.maximum(rowmax, jnp.max(sc, axis=1, keepdims=True))
        rowsum = jnp.zeros((U, 1), F32)
        upd = jnp.zeros((U, D), F32)
        for c in range(NKC):
            pr = jnp.exp(s_ref[:, c * KC:(c + 1) * KC] - rowmax)
            rowsum = rowsum + jnp.sum(pr, axis=1, keepdims=True)
            upd = upd + jax.lax.dot_general(
                pr, v_ref[c * KC:(c + 1) * KC, hh * D:(hh + 1) * D],
                (((1,), (0,)), ((), ())), preferred_element_type=F32)
        upds.append(upd / rowsum)

    acc = jnp.zeros((1, 2 * D), F32)
    for c in range(NKC):
        acc = acc + jnp.sum(v_ref[c * KC:(c + 1) * KC, :], axis=0,
                            keepdims=True)
    meanv = acc * (1.0 / L)
    for c in range(NKC):
        out_ref[c * KC:(c + 1) * KC, :] = jnp.broadcast_to(meanv, (KC, 2 * D))
    for hh in range(2):
        for i in range(U):
            row = out_ref[pl.ds(idxs_pair[hh][i], 1), :]
            ui = upds[hh][i:i + 1, :]
            if hh == 0:
                new = jnp.concatenate([ui, row[:, D:]], axis=1)
            else:
                new = jnp.concatenate([row[:, :D], ui], axis=1)
            out_ref[pl.ds(idxs_pair[hh][i], 1), :] = new


def kernel(queries, keys, values, attn_mask):
    B, Lq, Hh, Dd = queries.shape
    qf = queries.reshape(Lq, Hh * Dd)
    kf = keys.reshape(Lq, Hh * Dd)
    vf = values.reshape(Lq, Hh * Dd)

    wmask = jnp.asarray(_WMASK)
    wcnt = jnp.asarray(_WCNT)

    sel = pl.pallas_call(
        _m_kernel,
        grid=(Hh // 2,),
        in_specs=[
            pl.BlockSpec((L, 2 * D), lambda p: (0, p)),
            pl.BlockSpec((L, 2 * D), lambda p: (0, p)),
            pl.BlockSpec((L, L), lambda p: (0, 0)),
            pl.BlockSpec((L, L), lambda p: (0, 0)),
        ],
        out_specs=pl.BlockSpec((Hh, 128), lambda p: (0, 0)),
        out_shape=jax.ShapeDtypeStruct((Hh, 128), jnp.int32),
        scratch_shapes=[pltpu.VMEM((Hh, L), F32),
                        pltpu.VMEM((2, L, D), F32)],
    )(qf, kf, wmask, wcnt)

    ctx = pl.pallas_call(
        _attn_kernel,
        grid=(Hh // 2,),
        in_specs=[
            pl.BlockSpec(memory_space=pltpu.SMEM),
            pl.BlockSpec((L, 2 * D), lambda p: (0, p)),
            pl.BlockSpec((L, 2 * D), lambda p: (0, p)),
            pl.BlockSpec((L, 2 * D), lambda p: (0, p)),
        ],
        out_specs=pl.BlockSpec((L, 2 * D), lambda p: (0, p)),
        out_shape=jax.ShapeDtypeStruct((L, Hh * Dd), F32),
        scratch_shapes=[
            pltpu.VMEM((U, D), F32),
            pltpu.VMEM((U, L), F32),
        ],
    )(sel, qf, kf, vf)

    return ctx.reshape(B, Lq, Hh, Dd)
